# baseline (speedup 1.0000x reference)
import functools

import jax
import jax.numpy as jnp
from jax import lax
from jax.experimental import pallas as pl
from jax.experimental.pallas import tpu as pltpu
from jax.experimental.pallas import tpu_sc as plsc

RQ = 2.0
THRESHOLD = 4.0
R2 = (2.0 * RQ) ** 2

NQ = 720
TBLK = 2048

NC, NS = 2, 16
NW = NC * NS
N_SC = 4096
N_W = N_SC // NW
QG = NQ // 16
QB = 9


def _tc_body(q_ref, tT_ref, dsum_ref, cnt_ref, dacc, cacc):
    i = pl.program_id(0)
    nsteps = pl.num_programs(0)

    @pl.when(i == 0)
    def _init():
        dacc[...] = jnp.zeros_like(dacc)
        cacc[...] = jnp.zeros_like(cacc)

    g = jax.lax.dot_general(
        q_ref[...], tT_ref[...], (((1,), (0,)), ((), ())),
        preferred_element_type=jnp.float32)
    x = jnp.maximum(g, 1e-12)
    dist = x * jax.lax.rsqrt(x)
    m = x <= R2
    dist_m = jnp.where(m, dist, 0.0)
    m_f = jnp.where(m, 1.0, 0.0)

    def lane_tree_sum(a):
        cols = [a[:, k * 128:(k + 1) * 128] for k in range(TBLK // 128)]
        while len(cols) > 1:
            cols = [cols[j] + cols[j + 1] for j in range(0, len(cols), 2)]
        return cols[0]

    dacc[...] += lane_tree_sum(dist_m)
    cacc[...] += lane_tree_sum(m_f)

    @pl.when(i == nsteps - 1)
    def _fini():
        dsum_ref[...] = dacc[...].sum(axis=1, keepdims=True)
        cnt_ref[...] = cacc[...].sum(axis=1, keepdims=True)


def _tc_part(q, tT, nsteps):
    return pl.pallas_call(
        _tc_body,
        grid=(nsteps,),
        in_specs=[
            pl.BlockSpec((NQ, 5), lambda i: (0, 0)),
            pl.BlockSpec((5, TBLK), lambda i: (0, i)),
        ],
        out_specs=[
            pl.BlockSpec((NQ, 1), lambda i: (0, 0)),
            pl.BlockSpec((NQ, 1), lambda i: (0, 0)),
        ],
        out_shape=[
            jax.ShapeDtypeStruct((NQ, 1), jnp.float32),
            jax.ShapeDtypeStruct((NQ, 1), jnp.float32),
        ],
        scratch_shapes=[
            pltpu.VMEM((NQ, 128), jnp.float32),
            pltpu.VMEM((NQ, 128), jnp.float32),
        ],
    )(q, tT)


def _sc_body(q_hbm, t_hbm, dsum_hbm, cnt_hbm,
             qx_v, qy_v, qz_v, tx_v, ty_v, tz_v, dsum_v, cnt_v):
    c = lax.axis_index("c")
    s = lax.axis_index("s")
    w = s * NC + c
    base = w * N_W
    pltpu.sync_copy(q_hbm.at[pl.ds(0, NQ)], qx_v)
    pltpu.sync_copy(q_hbm.at[pl.ds(NQ, NQ)], qy_v)
    pltpu.sync_copy(q_hbm.at[pl.ds(2 * NQ, NQ)], qz_v)
    pltpu.sync_copy(t_hbm.at[pl.ds(base, N_W)], tx_v)
    pltpu.sync_copy(t_hbm.at[pl.ds(N_SC + base, N_W)], ty_v)
    pltpu.sync_copy(t_hbm.at[pl.ds(2 * N_SC + base, N_W)], tz_v)

    def blk_body(blk, _):
        qx = [qx_v[pl.ds((blk * QB + g) * 16, 16)] for g in range(QB)]
        qy = [qy_v[pl.ds((blk * QB + g) * 16, 16)] for g in range(QB)]
        qz = [qz_v[pl.ds((blk * QB + g) * 16, 16)] for g in range(QB)]

        def pt_body(k, carry):
            accd, accc = carry
            tx16 = tx_v[pl.ds(k * 16, 16)]
            ty16 = ty_v[pl.ds(k * 16, 16)]
            tz16 = tz_v[pl.ds(k * 16, 16)]
            for lane in range(16):
                tx = jnp.full((16,), tx16[lane], jnp.float32)
                ty = jnp.full((16,), ty16[lane], jnp.float32)
                tz = jnp.full((16,), tz16[lane], jnp.float32)
                for g2 in range(QB):
                    dx = qx[g2] - tx
                    dy = qy[g2] - ty
                    dz = qz[g2] - tz
                    x = dx * dx + dy * dy + dz * dz + 1e-12
                    bits = lax.bitcast_convert_type(x, jnp.int32)
                    y0 = lax.bitcast_convert_type(
                        jnp.int32(0x5F3759DF)
                        - lax.shift_right_logical(bits, 1),
                        jnp.float32)
                    y0 = y0 * (1.5 - 0.5 * x * y0 * y0)
                    dist = x * y0
                    m = x <= R2
                    accd[g2] = accd[g2] + jnp.where(m, dist, 0.0)
                    accc[g2] = accc[g2] + jnp.where(m, 1.0, 0.0)
            return accd, accc

        zeros = [jnp.zeros((16,), jnp.float32) for _ in range(QB)]
        accd, accc = lax.fori_loop(0, N_W // 16, pt_body,
                                   (zeros, list(zeros)))
        for g in range(QB):
            off = (blk * QB + g) * 16
            dsum_v[pl.ds(off, 16)] = accd[g]
            cnt_v[pl.ds(off, 16)] = accc[g]
        return 0

    lax.fori_loop(0, QG // QB, blk_body, 0)

    pltpu.sync_copy(dsum_v, dsum_hbm.at[pl.ds(w * NQ, NQ)])
    pltpu.sync_copy(cnt_v, cnt_hbm.at[pl.ds(w * NQ, NQ)])


def _sc_part(q_flat, t_flat):
    mesh = plsc.VectorSubcoreMesh(core_axis_name="c", subcore_axis_name="s")
    f = functools.partial(
        pl.kernel,
        out_type=[
            jax.ShapeDtypeStruct((NW * NQ,), jnp.float32),
            jax.ShapeDtypeStruct((NW * NQ,), jnp.float32),
        ],
        mesh=mesh,
        scratch_types=[
            pltpu.VMEM((NQ,), jnp.float32),
            pltpu.VMEM((NQ,), jnp.float32),
            pltpu.VMEM((NQ,), jnp.float32),
            pltpu.VMEM((N_W,), jnp.float32),
            pltpu.VMEM((N_W,), jnp.float32),
            pltpu.VMEM((N_W,), jnp.float32),
            pltpu.VMEM((NQ,), jnp.float32),
            pltpu.VMEM((NQ,), jnp.float32),
        ],
    )(_sc_body)
    return f(q_flat, t_flat)


def kernel(predicted_trajectories_global, terrain_points):
    traj = predicted_trajectories_global
    B, P, T, D = traj.shape
    qpts = traj.reshape(-1, D)
    ones = jnp.ones((NQ, 1), jnp.float32)
    q2 = jnp.sum(qpts * qpts, axis=1, keepdims=True) + 1e-12
    q = jnp.concatenate([q2, -2.0 * qpts, ones], axis=1)

    n = terrain_points.shape[0]
    n_tc = n - N_SC
    t_sc = terrain_points[n_tc:]

    npad = ((n_tc + TBLK - 1) // TBLK) * TBLK
    t = jnp.pad(terrain_points[:n_tc], ((0, npad - n_tc), (0, 0)),
                constant_values=1e6)
    tT = jnp.concatenate(
        [jnp.ones((1, npad), jnp.float32), t.T,
         jnp.sum(t * t, axis=1)[None, :]], axis=0)

    dsum_sc, cnt_sc = _sc_part(qpts.T.reshape(-1), t_sc.T.reshape(-1))
    dsum_tc, cnt_tc = _tc_part(q, tT, npad // TBLK)

    dsum = dsum_tc[:, 0] + dsum_sc.reshape(NW, NQ).sum(axis=0)
    cnt = cnt_tc[:, 0] + cnt_sc.reshape(NW, NQ).sum(axis=0)
    d_mean = dsum / jnp.maximum(cnt, 1.0)
    cost = jnp.where(cnt > 0.0,
                     -(d_mean * d_mean) * (1.0 / (RQ * RQ)) + THRESHOLD,
                     0.0)
    return cost.reshape(B, P, T).sum(axis=-1)


# speedup vs baseline: 1.0410x; 1.0410x over previous
import jax
import jax.numpy as jnp
from jax.experimental import pallas as pl
from jax.experimental.pallas import tpu as pltpu

RQ = 2.0
THRESHOLD = 4.0
R2 = (2.0 * RQ) ** 2

TBLK = 2048
NQ = 720


def _body(q_ref, tT_ref, cost_ref, dsum_ref, cnt_ref):
    i = pl.program_id(0)
    nsteps = pl.num_programs(0)

    @pl.when(i == 0)
    def _init():
        dsum_ref[...] = jnp.zeros_like(dsum_ref)
        cnt_ref[...] = jnp.zeros_like(cnt_ref)

    g = jax.lax.dot_general(
        q_ref[...], tT_ref[...], (((1,), (1,)), ((), ())),
        preferred_element_type=jnp.float32)
    x = jnp.maximum(g, 1e-12)
    dist = x * jax.lax.rsqrt(x)
    m = x <= R2
    dist_m = jnp.where(m, dist, 0.0)
    m_f = jnp.where(m, 1.0, 0.0)

    def lane_tree_sum(a):
        cols = [a[:, k * 128:(k + 1) * 128] for k in range(TBLK // 128)]
        while len(cols) > 1:
            cols = [cols[j] + cols[j + 1] for j in range(0, len(cols), 2)]
        return cols[0]

    dsum_ref[...] += lane_tree_sum(dist_m)
    cnt_ref[...] += lane_tree_sum(m_f)

    @pl.when(i == nsteps - 1)
    def _fini():
        cnt = cnt_ref[...].sum(axis=1, keepdims=True)
        dsum = dsum_ref[...].sum(axis=1, keepdims=True)
        d_mean = dsum / jnp.maximum(cnt, 1.0)
        cost = -(d_mean * d_mean) * (1.0 / (RQ * RQ)) + THRESHOLD
        cost_ref[...] = jnp.where(cnt > 0.0, cost, 0.0)


def kernel(predicted_trajectories_global, terrain_points):
    traj = predicted_trajectories_global
    B, P, T, D = traj.shape
    qpts = traj.reshape(-1, D)
    ones = jnp.ones((NQ, 1), jnp.float32)
    q2 = jnp.sum(qpts * qpts, axis=1, keepdims=True) + 1e-12
    q = jnp.concatenate([q2, -2.0 * qpts, ones], axis=1)

    n = terrain_points.shape[0]
    npad = ((n + TBLK - 1) // TBLK) * TBLK
    t = jnp.pad(terrain_points, ((0, npad - n), (0, 0)),
                constant_values=1e6)
    tT = jnp.concatenate(
        [jnp.ones((npad, 1), jnp.float32), t,
         jnp.sum(t * t, axis=1, keepdims=True)], axis=1)

    nsteps = npad // TBLK
    cost = pl.pallas_call(
        _body,
        grid=(nsteps,),
        in_specs=[
            pl.BlockSpec((NQ, 5), lambda i: (0, 0)),
            pl.BlockSpec((TBLK, 5), lambda i: (i, 0)),
        ],
        out_specs=pl.BlockSpec((NQ, 1), lambda i: (0, 0)),
        out_shape=jax.ShapeDtypeStruct((NQ, 1), jnp.float32),
        scratch_shapes=[
            pltpu.VMEM((NQ, 128), jnp.float32),
            pltpu.VMEM((NQ, 128), jnp.float32),
        ],
    )(q, tT)

    return cost.reshape(B, P, T).sum(axis=-1)


# speedup vs baseline: 1.3677x; 1.3138x over previous
import jax
import jax.numpy as jnp
from jax.experimental import pallas as pl
from jax.experimental.pallas import tpu as pltpu

RQ = 2.0
THRESHOLD = 4.0
R2 = (2.0 * RQ) ** 2

TBLK = 2048
NQ = 720


def _body(q_ref, tT_ref, cost_ref, dsum_ref, cnt_ref):
    i = pl.program_id(0)
    nsteps = pl.num_programs(0)

    @pl.when(i == 0)
    def _init():
        dsum_ref[...] = jnp.zeros_like(dsum_ref)
        cnt_ref[...] = jnp.zeros_like(cnt_ref)

    g = jax.lax.dot_general(
        q_ref[...], tT_ref[...], (((1,), (0,)), ((), ())),
        preferred_element_type=jnp.float32)
    x = jnp.maximum(g, 1e-12)
    dist = x * jax.lax.rsqrt(x)
    m = x <= R2
    dist_m = jnp.where(m, dist, 0.0)
    m_f = jnp.where(m, 1.0, 0.0)

    def lane_tree_sum(a):
        cols = [a[:, k * 128:(k + 1) * 128] for k in range(TBLK // 128)]
        while len(cols) > 1:
            cols = [cols[j] + cols[j + 1] for j in range(0, len(cols), 2)]
        return cols[0]

    dsum_ref[...] += lane_tree_sum(dist_m)
    cnt_ref[...] += lane_tree_sum(m_f)

    @pl.when(i == nsteps - 1)
    def _fini():
        cnt = cnt_ref[...].sum(axis=1, keepdims=True)
        dsum = dsum_ref[...].sum(axis=1, keepdims=True)
        d_mean = dsum / jnp.maximum(cnt, 1.0)
        cost = -(d_mean * d_mean) * (1.0 / (RQ * RQ)) + THRESHOLD
        cost_ref[...] = jnp.where(cnt > 0.0, cost, 0.0)


def kernel(predicted_trajectories_global, terrain_points):
    traj = predicted_trajectories_global
    B, P, T, D = traj.shape
    qpts = traj.reshape(-1, D)
    ones = jnp.ones((NQ, 1), jnp.float32)
    q2 = jnp.sum(qpts * qpts, axis=1, keepdims=True) + 1e-12
    q = jnp.concatenate([q2, -2.0 * qpts, ones], axis=1)

    n = terrain_points.shape[0]
    npad = ((n + TBLK - 1) // TBLK) * TBLK
    t = jnp.pad(terrain_points, ((0, npad - n), (0, 0)),
                constant_values=1e6)
    tT = jnp.concatenate(
        [jnp.ones((1, npad), jnp.float32), t.T,
         jnp.sum(t * t, axis=1)[None, :]], axis=0)

    nsteps = npad // TBLK
    cost = pl.pallas_call(
        _body,
        grid=(nsteps,),
        in_specs=[
            pl.BlockSpec((NQ, 5), lambda i: (0, 0)),
            pl.BlockSpec((5, TBLK), lambda i: (0, i)),
        ],
        out_specs=pl.BlockSpec((NQ, 1), lambda i: (0, 0)),
        out_shape=jax.ShapeDtypeStruct((NQ, 1), jnp.float32),
        scratch_shapes=[
            pltpu.VMEM((NQ, 128), jnp.float32),
            pltpu.VMEM((NQ, 128), jnp.float32),
        ],
    )(q, tT)

    return cost.reshape(B, P, T).sum(axis=-1)


# speedup vs baseline: 1.3921x; 1.0179x over previous
import jax
import jax.numpy as jnp
from jax.experimental import pallas as pl
from jax.experimental.pallas import tpu as pltpu

RQ = 2.0
THRESHOLD = 4.0
R2 = (2.0 * RQ) ** 2

TBLK = 2560
NQ = 720


def _body(q_ref, tT_ref, cost_ref, dsum_ref, cnt_ref):
    i = pl.program_id(0)
    nsteps = pl.num_programs(0)

    @pl.when(i == 0)
    def _init():
        dsum_ref[...] = jnp.zeros_like(dsum_ref)
        cnt_ref[...] = jnp.zeros_like(cnt_ref)

    g = jax.lax.dot_general(
        q_ref[...], tT_ref[...], (((1,), (0,)), ((), ())),
        preferred_element_type=jnp.float32)
    x = jnp.maximum(g, 1e-12)
    dist = x * jax.lax.rsqrt(x)
    m = x <= R2
    dist_m = jnp.where(m, dist, 0.0)
    m_f = jnp.where(m, 1.0, 0.0)

    def lane_tree_sum(a):
        cols = [a[:, k * 128:(k + 1) * 128] for k in range(TBLK // 128)]
        while len(cols) > 1:
            nxt = [cols[j] + cols[j + 1] for j in range(0, len(cols) - 1, 2)]
            if len(cols) % 2:
                nxt.append(cols[-1])
            cols = nxt
        return cols[0]

    dsum_ref[...] += lane_tree_sum(dist_m)
    cnt_ref[...] += lane_tree_sum(m_f)

    @pl.when(i == nsteps - 1)
    def _fini():
        cnt = cnt_ref[...].sum(axis=1, keepdims=True)
        dsum = dsum_ref[...].sum(axis=1, keepdims=True)
        d_mean = dsum / jnp.maximum(cnt, 1.0)
        cost = -(d_mean * d_mean) * (1.0 / (RQ * RQ)) + THRESHOLD
        cost_ref[...] = jnp.where(cnt > 0.0, cost, 0.0)


def kernel(predicted_trajectories_global, terrain_points):
    traj = predicted_trajectories_global
    B, P, T, D = traj.shape
    qpts = traj.reshape(-1, D)
    ones = jnp.ones((NQ, 1), jnp.float32)
    q2 = jnp.sum(qpts * qpts, axis=1, keepdims=True) + 1e-12
    q = jnp.concatenate([q2, -2.0 * qpts, ones], axis=1)

    n = terrain_points.shape[0]
    npad = ((n + TBLK - 1) // TBLK) * TBLK
    t = jnp.pad(terrain_points, ((0, npad - n), (0, 0)),
                constant_values=1e6)
    tT = jnp.concatenate(
        [jnp.ones((1, npad), jnp.float32), t.T,
         jnp.sum(t * t, axis=1)[None, :]], axis=0)

    nsteps = npad // TBLK
    cost = pl.pallas_call(
        _body,
        grid=(nsteps,),
        in_specs=[
            pl.BlockSpec((NQ, 5), lambda i: (0, 0)),
            pl.BlockSpec((5, TBLK), lambda i: (0, i)),
        ],
        out_specs=pl.BlockSpec((NQ, 1), lambda i: (0, 0)),
        out_shape=jax.ShapeDtypeStruct((NQ, 1), jnp.float32),
        scratch_shapes=[
            pltpu.VMEM((NQ, 128), jnp.float32),
            pltpu.VMEM((NQ, 128), jnp.float32),
        ],
    )(q, tT)

    return cost.reshape(B, P, T).sum(axis=-1)


# speedup vs baseline: 1.3981x; 1.0043x over previous
import jax
import jax.numpy as jnp
from jax.experimental import pallas as pl
from jax.experimental.pallas import tpu as pltpu

RQ = 2.0
THRESHOLD = 4.0
R2 = (2.0 * RQ) ** 2

TBLK = 3200
NQ = 720


def _body(q_ref, tT_ref, cost_ref, dsum_ref, cnt_ref):
    i = pl.program_id(0)
    nsteps = pl.num_programs(0)

    @pl.when(i == 0)
    def _init():
        dsum_ref[...] = jnp.zeros_like(dsum_ref)
        cnt_ref[...] = jnp.zeros_like(cnt_ref)

    g = jax.lax.dot_general(
        q_ref[...], tT_ref[...], (((1,), (0,)), ((), ())),
        preferred_element_type=jnp.float32)
    x = jnp.maximum(g, 1e-12)
    dist = x * jax.lax.rsqrt(x)
    m = x <= R2
    dist_m = jnp.where(m, dist, 0.0)
    m_f = jnp.where(m, 1.0, 0.0)

    def lane_tree_sum(a):
        cols = [a[:, k * 128:(k + 1) * 128] for k in range(TBLK // 128)]
        while len(cols) > 1:
            nxt = [cols[j] + cols[j + 1] for j in range(0, len(cols) - 1, 2)]
            if len(cols) % 2:
                nxt.append(cols[-1])
            cols = nxt
        return cols[0]

    dsum_ref[...] += lane_tree_sum(dist_m)
    cnt_ref[...] += lane_tree_sum(m_f)

    @pl.when(i == nsteps - 1)
    def _fini():
        cnt = cnt_ref[...].sum(axis=1, keepdims=True)
        dsum = dsum_ref[...].sum(axis=1, keepdims=True)
        d_mean = dsum / jnp.maximum(cnt, 1.0)
        cost = -(d_mean * d_mean) * (1.0 / (RQ * RQ)) + THRESHOLD
        cost_ref[...] = jnp.where(cnt > 0.0, cost, 0.0)


def kernel(predicted_trajectories_global, terrain_points):
    traj = predicted_trajectories_global
    B, P, T, D = traj.shape
    qpts = traj.reshape(-1, D)
    ones = jnp.ones((NQ, 1), jnp.float32)
    q2 = jnp.sum(qpts * qpts, axis=1, keepdims=True) + 1e-12
    q = jnp.concatenate([q2, -2.0 * qpts, ones], axis=1)

    n = terrain_points.shape[0]
    npad = ((n + TBLK - 1) // TBLK) * TBLK
    t = jnp.pad(terrain_points, ((0, npad - n), (0, 0)),
                constant_values=1e6)
    tT = jnp.concatenate(
        [jnp.ones((1, npad), jnp.float32), t.T,
         jnp.sum(t * t, axis=1)[None, :]], axis=0)

    nsteps = npad // TBLK
    cost = pl.pallas_call(
        _body,
        grid=(nsteps,),
        in_specs=[
            pl.BlockSpec((NQ, 5), lambda i: (0, 0)),
            pl.BlockSpec((5, TBLK), lambda i: (0, i)),
        ],
        out_specs=pl.BlockSpec((NQ, 1), lambda i: (0, 0)),
        out_shape=jax.ShapeDtypeStruct((NQ, 1), jnp.float32),
        scratch_shapes=[
            pltpu.VMEM((NQ, 128), jnp.float32),
            pltpu.VMEM((NQ, 128), jnp.float32),
        ],
    )(q, tT)

    return cost.reshape(B, P, T).sum(axis=-1)


# speedup vs baseline: 1.4443x; 1.0330x over previous
import jax
import jax.numpy as jnp
from jax.experimental import pallas as pl
from jax.experimental.pallas import tpu as pltpu

RQ = 2.0
THRESHOLD = 4.0
R2 = (2.0 * RQ) ** 2

TBLK = 5120
NQ = 720


def _body(q_ref, tT_ref, cost_ref, dsum_ref, cnt_ref):
    i = pl.program_id(0)
    nsteps = pl.num_programs(0)

    @pl.when(i == 0)
    def _init():
        dsum_ref[...] = jnp.zeros_like(dsum_ref)
        cnt_ref[...] = jnp.zeros_like(cnt_ref)

    g = jax.lax.dot_general(
        q_ref[...], tT_ref[...], (((1,), (0,)), ((), ())),
        preferred_element_type=jnp.float32)
    x = jnp.maximum(g, 1e-12)
    dist = x * jax.lax.rsqrt(x)
    m = x <= R2
    dist_m = jnp.where(m, dist, 0.0)
    m_f = jnp.where(m, 1.0, 0.0)

    def lane_tree_sum(a):
        cols = [a[:, k * 128:(k + 1) * 128] for k in range(TBLK // 128)]
        while len(cols) > 1:
            nxt = [cols[j] + cols[j + 1] for j in range(0, len(cols) - 1, 2)]
            if len(cols) % 2:
                nxt.append(cols[-1])
            cols = nxt
        return cols[0]

    dsum_ref[...] += lane_tree_sum(dist_m)
    cnt_ref[...] += lane_tree_sum(m_f)

    @pl.when(i == nsteps - 1)
    def _fini():
        cnt = cnt_ref[...].sum(axis=1, keepdims=True)
        dsum = dsum_ref[...].sum(axis=1, keepdims=True)
        d_mean = dsum / jnp.maximum(cnt, 1.0)
        cost = -(d_mean * d_mean) * (1.0 / (RQ * RQ)) + THRESHOLD
        cost_ref[...] = jnp.where(cnt > 0.0, cost, 0.0)


def kernel(predicted_trajectories_global, terrain_points):
    traj = predicted_trajectories_global
    B, P, T, D = traj.shape
    qpts = traj.reshape(-1, D)
    ones = jnp.ones((NQ, 1), jnp.float32)
    q2 = jnp.sum(qpts * qpts, axis=1, keepdims=True) + 1e-12
    q = jnp.concatenate([q2, -2.0 * qpts, ones], axis=1)

    n = terrain_points.shape[0]
    npad = ((n + TBLK - 1) // TBLK) * TBLK
    t = jnp.pad(terrain_points, ((0, npad - n), (0, 0)),
                constant_values=1e6)
    tT = jnp.concatenate(
        [jnp.ones((1, npad), jnp.float32), t.T,
         jnp.sum(t * t, axis=1)[None, :]], axis=0)

    nsteps = npad // TBLK
    cost = pl.pallas_call(
        _body,
        grid=(nsteps,),
        in_specs=[
            pl.BlockSpec((NQ, 5), lambda i: (0, 0)),
            pl.BlockSpec((5, TBLK), lambda i: (0, i)),
        ],
        out_specs=pl.BlockSpec((NQ, 1), lambda i: (0, 0)),
        out_shape=jax.ShapeDtypeStruct((NQ, 1), jnp.float32),
        scratch_shapes=[
            pltpu.VMEM((NQ, 128), jnp.float32),
            pltpu.VMEM((NQ, 128), jnp.float32),
        ],
    )(q, tT)

    return cost.reshape(B, P, T).sum(axis=-1)


# speedup vs baseline: 1.4682x; 1.0166x over previous
import jax
import jax.numpy as jnp
from jax.experimental import pallas as pl
from jax.experimental.pallas import tpu as pltpu

RQ = 2.0
THRESHOLD = 4.0
R2 = (2.0 * RQ) ** 2

TBLK = 10240
NQ = 720


def _body(q_ref, tT_ref, cost_ref, dsum_ref, cnt_ref):
    i = pl.program_id(0)
    nsteps = pl.num_programs(0)

    @pl.when(i == 0)
    def _init():
        dsum_ref[...] = jnp.zeros_like(dsum_ref)
        cnt_ref[...] = jnp.zeros_like(cnt_ref)

    g = jax.lax.dot_general(
        q_ref[...], tT_ref[...], (((1,), (0,)), ((), ())),
        preferred_element_type=jnp.float32)
    x = jnp.maximum(g, 1e-12)
    dist = x * jax.lax.rsqrt(x)
    m = x <= R2
    dist_m = jnp.where(m, dist, 0.0)
    m_f = jnp.where(m, 1.0, 0.0)

    def lane_tree_sum(a):
        cols = [a[:, k * 128:(k + 1) * 128] for k in range(TBLK // 128)]
        while len(cols) > 1:
            nxt = [cols[j] + cols[j + 1] for j in range(0, len(cols) - 1, 2)]
            if len(cols) % 2:
                nxt.append(cols[-1])
            cols = nxt
        return cols[0]

    dsum_ref[...] += lane_tree_sum(dist_m)
    cnt_ref[...] += lane_tree_sum(m_f)

    @pl.when(i == nsteps - 1)
    def _fini():
        cnt = cnt_ref[...].sum(axis=1, keepdims=True)
        dsum = dsum_ref[...].sum(axis=1, keepdims=True)
        d_mean = dsum / jnp.maximum(cnt, 1.0)
        cost = -(d_mean * d_mean) * (1.0 / (RQ * RQ)) + THRESHOLD
        cost_ref[...] = jnp.where(cnt > 0.0, cost, 0.0)


def kernel(predicted_trajectories_global, terrain_points):
    traj = predicted_trajectories_global
    B, P, T, D = traj.shape
    qpts = traj.reshape(-1, D)
    ones = jnp.ones((NQ, 1), jnp.float32)
    q2 = jnp.sum(qpts * qpts, axis=1, keepdims=True) + 1e-12
    q = jnp.concatenate([q2, -2.0 * qpts, ones], axis=1)

    n = terrain_points.shape[0]
    npad = ((n + TBLK - 1) // TBLK) * TBLK
    t = jnp.pad(terrain_points, ((0, npad - n), (0, 0)),
                constant_values=1e6)
    tT = jnp.concatenate(
        [jnp.ones((1, npad), jnp.float32), t.T,
         jnp.sum(t * t, axis=1)[None, :]], axis=0)

    nsteps = npad // TBLK
    cost = pl.pallas_call(
        _body,
        grid=(nsteps,),
        in_specs=[
            pl.BlockSpec((NQ, 5), lambda i: (0, 0)),
            pl.BlockSpec((5, TBLK), lambda i: (0, i)),
        ],
        out_specs=pl.BlockSpec((NQ, 1), lambda i: (0, 0)),
        out_shape=jax.ShapeDtypeStruct((NQ, 1), jnp.float32),
        scratch_shapes=[
            pltpu.VMEM((NQ, 128), jnp.float32),
            pltpu.VMEM((NQ, 128), jnp.float32),
        ],
    )(q, tT)

    return cost.reshape(B, P, T).sum(axis=-1)


# speedup vs baseline: 1.4819x; 1.0093x over previous
import jax
import jax.numpy as jnp
from jax.experimental import pallas as pl
from jax.experimental.pallas import tpu as pltpu

RQ = 2.0
THRESHOLD = 4.0
R2 = (2.0 * RQ) ** 2

TBLK = 25600
NQ = 720


def _body(q_ref, tT_ref, cost_ref, dsum_ref, cnt_ref):
    i = pl.program_id(0)
    nsteps = pl.num_programs(0)

    @pl.when(i == 0)
    def _init():
        dsum_ref[...] = jnp.zeros_like(dsum_ref)
        cnt_ref[...] = jnp.zeros_like(cnt_ref)

    g = jax.lax.dot_general(
        q_ref[...], tT_ref[...], (((1,), (0,)), ((), ())),
        preferred_element_type=jnp.float32)
    x = jnp.maximum(g, 1e-12)
    dist = x * jax.lax.rsqrt(x)
    m = x <= R2
    dist_m = jnp.where(m, dist, 0.0)
    m_f = jnp.where(m, 1.0, 0.0)

    def lane_tree_sum(a):
        cols = [a[:, k * 128:(k + 1) * 128] for k in range(TBLK // 128)]
        while len(cols) > 1:
            nxt = [cols[j] + cols[j + 1] for j in range(0, len(cols) - 1, 2)]
            if len(cols) % 2:
                nxt.append(cols[-1])
            cols = nxt
        return cols[0]

    dsum_ref[...] += lane_tree_sum(dist_m)
    cnt_ref[...] += lane_tree_sum(m_f)

    @pl.when(i == nsteps - 1)
    def _fini():
        cnt = cnt_ref[...].sum(axis=1, keepdims=True)
        dsum = dsum_ref[...].sum(axis=1, keepdims=True)
        d_mean = dsum / jnp.maximum(cnt, 1.0)
        cost = -(d_mean * d_mean) * (1.0 / (RQ * RQ)) + THRESHOLD
        cost_ref[...] = jnp.where(cnt > 0.0, cost, 0.0)


def kernel(predicted_trajectories_global, terrain_points):
    traj = predicted_trajectories_global
    B, P, T, D = traj.shape
    qpts = traj.reshape(-1, D)
    ones = jnp.ones((NQ, 1), jnp.float32)
    q2 = jnp.sum(qpts * qpts, axis=1, keepdims=True) + 1e-12
    q = jnp.concatenate([q2, -2.0 * qpts, ones], axis=1)

    n = terrain_points.shape[0]
    npad = ((n + TBLK - 1) // TBLK) * TBLK
    t = jnp.pad(terrain_points, ((0, npad - n), (0, 0)),
                constant_values=1e6)
    tT = jnp.concatenate(
        [jnp.ones((1, npad), jnp.float32), t.T,
         jnp.sum(t * t, axis=1)[None, :]], axis=0)

    nsteps = npad // TBLK
    cost = pl.pallas_call(
        _body,
        grid=(nsteps,),
        in_specs=[
            pl.BlockSpec((NQ, 5), lambda i: (0, 0)),
            pl.BlockSpec((5, TBLK), lambda i: (0, i)),
        ],
        out_specs=pl.BlockSpec((NQ, 1), lambda i: (0, 0)),
        out_shape=jax.ShapeDtypeStruct((NQ, 1), jnp.float32),
        scratch_shapes=[
            pltpu.VMEM((NQ, 128), jnp.float32),
            pltpu.VMEM((NQ, 128), jnp.float32),
        ],
    )(q, tT)

    return cost.reshape(B, P, T).sum(axis=-1)
